# baseline (speedup 1.0000x reference)
import functools

import jax
import jax.numpy as jnp
import numpy as np
from jax import lax
from jax.experimental import pallas as pl
from jax.experimental.pallas import tpu as pltpu
from jax.experimental.pallas import tpu_sc as plsc

BATCH = 16384
KNOW = 128
L = 16
NC, NS = 2, 16
NW = NC * NS
BPW = BATCH // NW
CH = 128
NCH = BPW // CH
T_INV = 1.0 / 50.0


def _body(user_hbm, item_hbm, kn_hbm, theta_hbm, slip_hbm, guess_hbm, out_hbm,
          idx_u, idx_i, sl_raw, gu_raw, th0, th1, kn0, kn1, out_v, tb,
          sem_a, sem_b):
    wid = lax.axis_index("s") * NC + lax.axis_index("c")
    base = wid * BPW

    pltpu.sync_copy(user_hbm.at[pl.ds(base, BPW)], idx_u)
    pltpu.sync_copy(item_hbm.at[pl.ds(base, BPW)], idx_i)

    th_bufs = (th0, th1)
    kn_bufs = (kn0, kn1)
    sems = (sem_a, sem_b)

    def fire(ch):
        slot = ch % 2
        ids = pl.ds(ch * CH, CH)
        sem = sems[slot]
        return (
            pltpu.async_copy(theta_hbm.at[idx_u.at[ids]], th_bufs[slot], sem),
            pltpu.async_copy(kn_hbm.at[pl.ds(base + ch * CH, CH)],
                             kn_bufs[slot], sem),
            pltpu.async_copy(slip_hbm.at[idx_i.at[ids]], sl_raw.at[ids], sem),
            pltpu.async_copy(guess_hbm.at[idx_i.at[ids]], gu_raw.at[ids], sem),
        )

    def compute(ch):
        slot = ch % 2
        th_ref = th_bufs[slot]
        kn_ref = kn_bufs[slot]

        def group(g, carry):
            lane = lax.iota(jnp.int32, L)
            for r in range(L):
                row = g * L + r
                acc = jnp.zeros((L,), jnp.float32)
                for cc in range(KNOW // L):
                    th = th_ref[row, pl.ds(cc * L, L)]
                    kn = kn_ref[row, pl.ds(cc * L, L)]
                    e = jnp.exp(-jnp.abs(th))
                    s = (1.0 - e) / (2.0 + 2.0 * e)
                    acc = acc + kn * (s * jnp.sign(th))
                tb[pl.ds(r * L, L)] = acc
            nvec = jnp.zeros((L,), jnp.float32)
            for j in range(L):
                nvec = nvec + plsc.load_gather(tb, [lane * L + j])
            off = ch * CH + g * L
            p = 1.0 / (1.0 + jnp.exp(-nvec * T_INV))
            slip = 0.4 / (1.0 + jnp.exp(-sl_raw[pl.ds(off, L)]))
            guess = 0.4 / (1.0 + jnp.exp(-gu_raw[pl.ds(off, L)]))
            out_v[pl.ds(off, L)] = guess + (1.0 - slip - guess) * p
            return carry

        lax.fori_loop(0, CH // L, group, 0)

    pending = fire(0)
    for ch in range(NCH):
        nxt = fire(ch + 1) if ch + 1 < NCH else ()
        for c in pending:
            c.wait()
        compute(ch)
        pending = nxt

    pltpu.sync_copy(out_v, out_hbm.at[pl.ds(base, BPW)])


@functools.partial(jax.jit, static_argnames=())
def _dina(user, item, knowledge, theta_table, slip_flat, guess_flat):
    run = pl.kernel(
        _body,
        out_type=jax.ShapeDtypeStruct((BATCH,), jnp.float32),
        mesh=plsc.VectorSubcoreMesh(core_axis_name="c", subcore_axis_name="s",
                                    num_cores=NC, num_subcores=NS),
        compiler_params=pltpu.CompilerParams(needs_layout_passes=False),
        scratch_types=[
            pltpu.VMEM((BPW,), jnp.int32),
            pltpu.VMEM((BPW,), jnp.int32),
            pltpu.VMEM((BPW,), jnp.float32),
            pltpu.VMEM((BPW,), jnp.float32),
            pltpu.VMEM((CH, KNOW), jnp.float32),
            pltpu.VMEM((CH, KNOW), jnp.float32),
            pltpu.VMEM((CH, KNOW), jnp.float32),
            pltpu.VMEM((CH, KNOW), jnp.float32),
            pltpu.VMEM((BPW,), jnp.float32),
            pltpu.VMEM((L * L,), jnp.float32),
            pltpu.SemaphoreType.DMA,
            pltpu.SemaphoreType.DMA,
        ],
        name="dina_sc",
    )
    return run(user, item, knowledge, theta_table, slip_flat, guess_flat)


def kernel(user, item, knowledge, theta_table, slip_table, guess_table):
    return _dina(user, item, knowledge, theta_table,
                 slip_table.reshape(-1), guess_table.reshape(-1))


# speedup vs baseline: 2.7324x; 2.7324x over previous
import functools

import jax
import jax.numpy as jnp
from jax import lax
from jax.experimental import pallas as pl
from jax.experimental.pallas import tpu as pltpu
from jax.experimental.pallas import tpu_sc as plsc

BATCH = 16384
KNOW = 128
NC, NS = 2, 16
NW = NC * NS
BPW = BATCH // NW
CH = 128
NCH = BPW // CH
T_INV = 1.0 / 50.0
TC_ROWS = 1024


def _gather_body(user_hbm, item_hbm, theta_hbm, slip_hbm, guess_hbm,
                 thg_hbm, sraw_hbm, graw_hbm,
                 idx_u, idx_i, t0, t1, s0, s1, g0, g1,
                 sem_i0, sem_i1, sem_o0, sem_o1):
    wid = lax.axis_index("s") * NC + lax.axis_index("c")
    base = wid * BPW

    pltpu.sync_copy(user_hbm.at[pl.ds(base, BPW)], idx_u)
    pltpu.sync_copy(item_hbm.at[pl.ds(base, BPW)], idx_i)

    tb, sb, gb = (t0, t1), (s0, s1), (g0, g1)
    sem_i, sem_o = (sem_i0, sem_i1), (sem_o0, sem_o1)

    def fire_in(ch):
        slot = ch % 2
        ids = pl.ds(ch * CH, CH)
        return (
            pltpu.async_copy(theta_hbm.at[idx_u.at[ids]], tb[slot], sem_i[slot]),
            pltpu.async_copy(slip_hbm.at[idx_i.at[ids]], sb[slot], sem_i[slot]),
            pltpu.async_copy(guess_hbm.at[idx_i.at[ids]], gb[slot], sem_i[slot]),
        )

    def fire_out(ch):
        slot = ch % 2
        ids = pl.ds(base + ch * CH, CH)
        return (
            pltpu.async_copy(tb[slot], thg_hbm.at[ids], sem_o[slot]),
            pltpu.async_copy(sb[slot], sraw_hbm.at[ids], sem_o[slot]),
            pltpu.async_copy(gb[slot], graw_hbm.at[ids], sem_o[slot]),
        )

    pend_in = {0: fire_in(0), 1: fire_in(1)}
    tail = []
    for ch in range(NCH):
        for c in pend_in[ch]:
            c.wait()
        out_cp = fire_out(ch)
        if ch + 2 < NCH:
            for c in out_cp:
                c.wait()
            pend_in[ch + 2] = fire_in(ch + 2)
        else:
            tail.append(out_cp)
    for out_cp in tail:
        for c in out_cp:
            c.wait()


@jax.jit
def _sc_gather(user, item, theta_table, slip_flat, guess_flat):
    run = pl.kernel(
        _gather_body,
        out_type=(
            jax.ShapeDtypeStruct((BATCH, KNOW), jnp.float32),
            jax.ShapeDtypeStruct((BATCH,), jnp.float32),
            jax.ShapeDtypeStruct((BATCH,), jnp.float32),
        ),
        mesh=plsc.VectorSubcoreMesh(core_axis_name="c", subcore_axis_name="s",
                                    num_cores=NC, num_subcores=NS),
        compiler_params=pltpu.CompilerParams(needs_layout_passes=False),
        scratch_types=[
            pltpu.VMEM((BPW,), jnp.int32),
            pltpu.VMEM((BPW,), jnp.int32),
            pltpu.VMEM((CH, KNOW), jnp.float32),
            pltpu.VMEM((CH, KNOW), jnp.float32),
            pltpu.VMEM((CH,), jnp.float32),
            pltpu.VMEM((CH,), jnp.float32),
            pltpu.VMEM((CH,), jnp.float32),
            pltpu.VMEM((CH,), jnp.float32),
            pltpu.SemaphoreType.DMA,
            pltpu.SemaphoreType.DMA,
            pltpu.SemaphoreType.DMA,
            pltpu.SemaphoreType.DMA,
        ],
        name="dina_sc_gather",
    )
    return run(user, item, theta_table, slip_flat, guess_flat)


def _combine_body(th_ref, kn_ref, sr_ref, gr_ref, o_ref):
    th = th_ref[...]
    kn = kn_ref[...]
    s = jax.nn.sigmoid(th) - 0.5
    n = jnp.sum(kn * s, axis=1)
    p = jax.nn.sigmoid(n * T_INV)
    slip = 0.4 * jax.nn.sigmoid(sr_ref[...])
    guess = 0.4 * jax.nn.sigmoid(gr_ref[...])
    o_ref[...] = guess + (1.0 - slip - guess) * p


@jax.jit
def _tc_combine(theta_g, knowledge, s_raw, g_raw):
    return pl.pallas_call(
        _combine_body,
        grid=(BATCH // TC_ROWS,),
        in_specs=[
            pl.BlockSpec((TC_ROWS, KNOW), lambda i: (i, 0)),
            pl.BlockSpec((TC_ROWS, KNOW), lambda i: (i, 0)),
            pl.BlockSpec((TC_ROWS,), lambda i: (i,)),
            pl.BlockSpec((TC_ROWS,), lambda i: (i,)),
        ],
        out_specs=pl.BlockSpec((TC_ROWS,), lambda i: (i,)),
        out_shape=jax.ShapeDtypeStruct((BATCH,), jnp.float32),
        name="dina_tc_combine",
    )(theta_g, knowledge, s_raw, g_raw)


def kernel(user, item, knowledge, theta_table, slip_table, guess_table):
    theta_g, s_raw, g_raw = _sc_gather(
        user, item, theta_table,
        slip_table.reshape(-1), guess_table.reshape(-1))
    return _tc_combine(theta_g, knowledge, s_raw, g_raw)


# speedup vs baseline: 3.2843x; 1.2020x over previous
import functools

import jax
import jax.numpy as jnp
from jax import lax
from jax.experimental import pallas as pl
from jax.experimental.pallas import tpu as pltpu
from jax.experimental.pallas import tpu_sc as plsc

BATCH = 16384
KNOW = 128
NC, NS = 2, 16
NW = NC * NS
BPW = BATCH // NW
CH = 128
NCH = BPW // CH
T_INV = 1.0 / 50.0
TC_ROWS = 2048


def _gather_body(user_hbm, item_hbm, theta_hbm, slip_hbm, guess_hbm,
                 thg_hbm, sraw_hbm, graw_hbm,
                 idx_u, idx_i, t0, t1, s0, s1, g0, g1,
                 sem_i0, sem_i1, sem_o0, sem_o1):
    wid = lax.axis_index("s") * NC + lax.axis_index("c")
    base = wid * BPW

    pltpu.sync_copy(user_hbm.at[pl.ds(base, BPW)], idx_u)
    pltpu.sync_copy(item_hbm.at[pl.ds(base, BPW)], idx_i)

    tb, sb, gb = (t0, t1), (s0, s1), (g0, g1)
    sem_i, sem_o = (sem_i0, sem_i1), (sem_o0, sem_o1)

    def fire_in(ch):
        slot = ch % 2
        ids = pl.ds(ch * CH, CH)
        return (
            pltpu.async_copy(theta_hbm.at[idx_u.at[ids]], tb[slot], sem_i[slot]),
            pltpu.async_copy(slip_hbm.at[idx_i.at[ids]], sb[slot], sem_i[slot]),
            pltpu.async_copy(guess_hbm.at[idx_i.at[ids]], gb[slot], sem_i[slot]),
        )

    def fire_out(ch):
        slot = ch % 2
        ids = pl.ds(base + ch * CH, CH)
        return (
            pltpu.async_copy(tb[slot], thg_hbm.at[ids], sem_o[slot]),
            pltpu.async_copy(sb[slot], sraw_hbm.at[ids], sem_o[slot]),
            pltpu.async_copy(gb[slot], graw_hbm.at[ids], sem_o[slot]),
        )

    pend_in = {0: fire_in(0), 1: fire_in(1)}
    tail = []
    for ch in range(NCH):
        for c in pend_in[ch]:
            c.wait()
        out_cp = fire_out(ch)
        if ch + 2 < NCH:
            for c in out_cp:
                c.wait()
            pend_in[ch + 2] = fire_in(ch + 2)
        else:
            tail.append(out_cp)
    for out_cp in tail:
        for c in out_cp:
            c.wait()


@jax.jit
def _sc_gather(user, item, theta_table, slip_flat, guess_flat):
    run = pl.kernel(
        _gather_body,
        out_type=(
            jax.ShapeDtypeStruct((BATCH, KNOW), jnp.float32),
            jax.ShapeDtypeStruct((BATCH,), jnp.float32),
            jax.ShapeDtypeStruct((BATCH,), jnp.float32),
        ),
        mesh=plsc.VectorSubcoreMesh(core_axis_name="c", subcore_axis_name="s",
                                    num_cores=NC, num_subcores=NS),
        compiler_params=pltpu.CompilerParams(needs_layout_passes=False),
        scratch_types=[
            pltpu.VMEM((BPW,), jnp.int32),
            pltpu.VMEM((BPW,), jnp.int32),
            pltpu.VMEM((CH, KNOW), jnp.float32),
            pltpu.VMEM((CH, KNOW), jnp.float32),
            pltpu.VMEM((CH,), jnp.float32),
            pltpu.VMEM((CH,), jnp.float32),
            pltpu.VMEM((CH,), jnp.float32),
            pltpu.VMEM((CH,), jnp.float32),
            pltpu.SemaphoreType.DMA,
            pltpu.SemaphoreType.DMA,
            pltpu.SemaphoreType.DMA,
            pltpu.SemaphoreType.DMA,
        ],
        name="dina_sc_gather",
    )
    return run(user, item, theta_table, slip_flat, guess_flat)


def _combine_body(th_ref, kn_ref, sr_ref, gr_ref, o_ref):
    th = th_ref[...]
    kn = kn_ref[...]
    s = 0.5 * jnp.tanh(0.5 * th)
    prod_t = lax.transpose(kn * s, (1, 0))
    n = jnp.sum(prod_t, axis=0)
    p = 0.5 * jnp.tanh((0.5 * T_INV) * n) + 0.5
    slip = 0.2 * jnp.tanh(0.5 * sr_ref[...]) + 0.2
    guess = 0.2 * jnp.tanh(0.5 * gr_ref[...]) + 0.2
    o_ref[...] = guess + (1.0 - slip - guess) * p


@jax.jit
def _tc_combine(theta_g, knowledge, s_raw, g_raw):
    return pl.pallas_call(
        _combine_body,
        grid=(BATCH // TC_ROWS,),
        in_specs=[
            pl.BlockSpec((TC_ROWS, KNOW), lambda i: (i, 0)),
            pl.BlockSpec((TC_ROWS, KNOW), lambda i: (i, 0)),
            pl.BlockSpec((TC_ROWS,), lambda i: (i,)),
            pl.BlockSpec((TC_ROWS,), lambda i: (i,)),
        ],
        out_specs=pl.BlockSpec((TC_ROWS,), lambda i: (i,)),
        out_shape=jax.ShapeDtypeStruct((BATCH,), jnp.float32),
        name="dina_tc_combine",
    )(theta_g, knowledge, s_raw, g_raw)


def kernel(user, item, knowledge, theta_table, slip_table, guess_table):
    theta_g, s_raw, g_raw = _sc_gather(
        user, item, theta_table,
        slip_table.reshape(-1), guess_table.reshape(-1))
    return _tc_combine(theta_g, knowledge, s_raw, g_raw)
